# baseline (speedup 1.0000x reference)
import jax
import jax.numpy as jnp
from jax import lax
from jax.experimental import pallas as pl
from jax.experimental.pallas import tpu as pltpu
from jax.experimental.pallas import tpu_sc as plsc

NUM_CORES = 2
NUM_SUBCORES = 16
NUM_WORKERS = NUM_CORES * NUM_SUBCORES
SC_TOKENS = 8192
TC_BLOCK = 512
PAD_VOCAB = 128


def _tc_body(tok_ref, table_ref, out_ref):
    tok = tok_ref[...]
    onehot = (tok == lax.broadcasted_iota(
        jnp.int32, (tok.shape[0], PAD_VOCAB), 1)).astype(jnp.float32)
    out_ref[...] = jnp.dot(onehot, table_ref[...],
                           precision=lax.Precision.HIGHEST,
                           preferred_element_type=jnp.float32)


def kernel(tokens, embed_table):
    num_tokens = tokens.shape[0]
    vocab, dim = embed_table.shape
    n_sc = SC_TOKENS
    n_tc = num_tokens - n_sc
    b_per_w = n_sc // NUM_WORKERS

    mesh = plsc.VectorSubcoreMesh(core_axis_name="c", subcore_axis_name="s")

    @jax.jit
    def run(tok, table):
        @pl.kernel(
            mesh=mesh,
            out_type=jax.ShapeDtypeStruct((num_tokens, dim), table.dtype),
            scratch_types=[
                pltpu.VMEM((b_per_w,), jnp.int32),
                pltpu.VMEM((b_per_w, dim), table.dtype),
            ],
        )
        def sc_gather(idx_hbm, table_hbm, out_hbm, idx_v, rows_v):
            wid = lax.axis_index("s") * NUM_CORES + lax.axis_index("c")
            base = wid * b_per_w
            pltpu.sync_copy(idx_hbm.at[pl.ds(base, b_per_w)], idx_v)
            pltpu.sync_copy(table_hbm.at[idx_v], rows_v)
            pltpu.sync_copy(rows_v, out_hbm.at[pl.ds(base, b_per_w)])

        sc_full = sc_gather(tok, table)

        table_pad = jnp.zeros((PAD_VOCAB, dim), table.dtype).at[:vocab].set(
            table)
        tc_out = pl.pallas_call(
            _tc_body,
            grid=(n_tc // TC_BLOCK,),
            in_specs=[
                pl.BlockSpec((TC_BLOCK, 1), lambda i: (i, 0)),
                pl.BlockSpec((PAD_VOCAB, dim), lambda i: (0, 0)),
            ],
            out_specs=pl.BlockSpec((TC_BLOCK, dim), lambda i: (i, 0)),
            out_shape=jax.ShapeDtypeStruct((n_tc, dim), table.dtype),
        )(tok[n_sc:].reshape(n_tc, 1), table_pad)

        return lax.dynamic_update_slice(sc_full, tc_out, (n_sc, 0))

    return run(tokens.astype(jnp.int32), embed_table)


# speedup vs baseline: 1.1436x; 1.1436x over previous
import jax
import jax.numpy as jnp
from jax import lax
from jax.experimental import pallas as pl
from jax.experimental.pallas import tpu as pltpu
from jax.experimental.pallas import tpu_sc as plsc

NUM_CORES = 2
NUM_SUBCORES = 16
NUM_WORKERS = NUM_CORES * NUM_SUBCORES


def kernel(tokens, embed_table):
    num_tokens = tokens.shape[0]
    dim = embed_table.shape[1]
    b_per_w = num_tokens // NUM_WORKERS

    mesh = plsc.VectorSubcoreMesh(core_axis_name="c", subcore_axis_name="s")

    @jax.jit
    def run(tok, table):
        @pl.kernel(
            mesh=mesh,
            out_type=jax.ShapeDtypeStruct((num_tokens, dim), table.dtype),
            scratch_types=[
                pltpu.VMEM((b_per_w,), jnp.int32),
                pltpu.VMEM((b_per_w, dim), table.dtype),
            ],
        )
        def sc_gather(idx_hbm, table_hbm, out_hbm, idx_v, rows_v):
            wid = lax.axis_index("s") * NUM_CORES + lax.axis_index("c")
            base = wid * b_per_w
            pltpu.sync_copy(idx_hbm.at[pl.ds(base, b_per_w)], idx_v)
            pltpu.sync_copy(table_hbm.at[idx_v], rows_v)
            pltpu.sync_copy(rows_v, out_hbm.at[pl.ds(base, b_per_w)])

        return sc_gather(tok, table)

    return run(tokens.astype(jnp.int32), embed_table)
